# Initial kernel scaffold; baseline (speedup 1.0000x reference)
import jax
import jax.numpy as jnp
from jax.experimental import pallas as pl


def kernel(anchors, scores, regs):
    raise NotImplementedError("write your pallas kernel here")


# speedup vs baseline: 20.4673x; 20.4673x over previous
import functools
import math

import jax
import jax.numpy as jnp
from jax import lax
from jax.experimental import pallas as pl
from jax.experimental.pallas import tpu as pltpu

_IMG_H = 1024.0
_IMG_W = 1024.0
_PRE = 2000
_POST = 1000
_PAD = 2048
_BLK = 128
_NBLK = _PAD // _BLK
_THRESH = 0.7
_MIN_SIZE = 16.0
_LOG_MAX = math.log(1000.0 / 16.0)
_NEG_INF = float("-inf")


def _nms_body(ax1, ay1, ax2, ay2, dx, dy, dw, dh, s_ref,
              px1, py1, px2, py2, kept,
              keep, areas):
    w = ax2[...] - ax1[...]
    h = ay2[...] - ay1[...]
    cx = ax1[...] + 0.5 * w
    cy = ay1[...] + 0.5 * h
    dwc = jnp.minimum(dw[...], _LOG_MAX)
    dhc = jnp.minimum(dh[...], _LOG_MAX)
    pcx = dx[...] * w + cx
    pcy = dy[...] * h + cy
    pw = jnp.exp(dwc) * w
    ph = jnp.exp(dhc) * h
    x1 = jnp.clip(pcx - 0.5 * pw, 0.0, _IMG_W)
    y1 = jnp.clip(pcy - 0.5 * ph, 0.0, _IMG_H)
    x2 = jnp.clip(pcx + 0.5 * pw, 0.0, _IMG_W)
    y2 = jnp.clip(pcy + 0.5 * ph, 0.0, _IMG_H)
    px1[...] = x1
    py1[...] = y1
    px2[...] = x2
    py2[...] = y2

    ws = x2 - x1
    hs = y2 - y1
    valid = (ws >= _MIN_SIZE) & (hs >= _MIN_SIZE)
    keep[...] = valid.astype(jnp.float32)
    areas[...] = jnp.maximum(ws, 0.0) * jnp.maximum(hs, 0.0)

    for b in range(_NBLK):
        base = b * _BLK
        width = _PAD - base

        tx1 = px1[:, base:]
        ty1 = py1[:, base:]
        tx2 = px2[:, base:]
        ty2 = py2[:, base:]
        tar = areas[:, base:]
        jcol = lax.broadcasted_iota(jnp.int32, (8, width), 1)
        bx1 = px1[:, base:base + _BLK]
        by1 = py1[:, base:base + _BLK]
        bx2 = px2[:, base:base + _BLK]
        by2 = py2[:, base:base + _BLK]
        bar = areas[:, base:base + _BLK]
        icol = lax.broadcasted_iota(jnp.int32, (8, _BLK), 1)

        def _col(block, i):
            return jnp.where(icol == i, block, 0.0).sum(axis=1, keepdims=True)

        def body(i, _, base=base, width=width, tx1=tx1, ty1=ty1,
                 tx2=tx2, ty2=ty2, tar=tar, jcol=jcol,
                 bx1=bx1, by1=by1, bx2=bx2, by2=by2, bar=bar):
            rx1 = _col(bx1, i)
            ry1 = _col(by1, i)
            rx2 = _col(bx2, i)
            ry2 = _col(by2, i)
            rar = _col(bar, i)
            gate = _col(keep[:, base:base + _BLK], i)

            xx1 = jnp.maximum(rx1, tx1)
            yy1 = jnp.maximum(ry1, ty1)
            xx2 = jnp.minimum(rx2, tx2)
            yy2 = jnp.minimum(ry2, ty2)
            inter = jnp.maximum(xx2 - xx1, 0.0) * jnp.maximum(yy2 - yy1, 0.0)
            denom = jnp.maximum(rar + tar - inter, 1e-9)
            sup = (inter > _THRESH * denom) & (jcol > i)
            supf = sup.astype(jnp.float32) * gate
            ktail = keep[:, base:]
            keep[:, base:] = ktail * (1.0 - supf)
            return 0

        lax.fori_loop(0, _BLK, body, 0)

    kept[...] = jnp.where(keep[...] > 0.5, s_ref[...], _NEG_INF)


@jax.jit
def kernel(anchors, scores, regs):
    B, N = scores.shape
    top_scores, top_idx = lax.top_k(scores, _PRE)
    ta = jnp.take_along_axis(anchors, top_idx[..., None], axis=1)
    tr = jnp.take_along_axis(regs, top_idx[..., None], axis=1)

    pad = _PAD - _PRE
    s = jnp.pad(top_scores, ((0, 0), (0, pad)), constant_values=_NEG_INF)
    ta = jnp.pad(ta, ((0, 0), (0, pad), (0, 0)))
    tr = jnp.pad(tr, ((0, 0), (0, pad), (0, 0)))

    shape = jax.ShapeDtypeStruct((B, _PAD), jnp.float32)
    px1, py1, px2, py2, kept = pl.pallas_call(
        _nms_body,
        out_shape=(shape,) * 5,
        scratch_shapes=[pltpu.VMEM((B, _PAD), jnp.float32)] * 2,
    )(ta[..., 0], ta[..., 1], ta[..., 2], ta[..., 3],
      tr[..., 0], tr[..., 1], tr[..., 2], tr[..., 3], s)

    top_vals, top = lax.top_k(kept, _POST)
    valid = (top_vals > _NEG_INF).astype(jnp.float32)
    gx1 = jnp.take_along_axis(px1, top, axis=1)
    gy1 = jnp.take_along_axis(py1, top, axis=1)
    gx2 = jnp.take_along_axis(px2, top, axis=1)
    gy2 = jnp.take_along_axis(py2, top, axis=1)
    boxes = jnp.stack([gx1, gy1, gx2, gy2], axis=-1)
    return boxes * valid[..., None]
